# Initial kernel scaffold; baseline (speedup 1.0000x reference)
import functools

import jax
import jax.numpy as jnp
from jax import lax
from jax.experimental import pallas as pl
from jax.experimental.pallas import tpu as pltpu
from jax.experimental.pallas import tpu_sc as plsc

N = 10000
E = 320000
D = 128
G = 16

NC = 2
NS = 16
NW = NC * NS

CH = 128
RPW = 79
E_PAD = NW * RPW * CH
NROW = NW * RPW
NP = 10016
ZPT = NP // NS
OPT = N // NS

BN = 400
NBLK = N // BN

_mesh = plsc.VectorSubcoreMesh(core_axis_name="c", subcore_axis_name="s")
f32 = jnp.float32
i32 = jnp.int32


def _zero_rows(rows_v, ncols):
    nv = ncols // 16

    def body(i, c):
        r = i // nv
        k = (i % nv) * 16
        rows_v[r, pl.ds(k, 16)] = jnp.zeros((16,), f32)
        return c

    lax.fori_loop(0, 128 * nv, body, 0)


def _fill_spmem(rows_v, part_s, base, nrows):
    full, rem = nrows // 128, nrows % 128
    for k in range(full):
        pltpu.sync_copy(rows_v, part_s.at[pl.ds(base + k * 128, 128)])
    if rem:
        pltpu.sync_copy(rows_v.at[pl.ds(0, rem)], part_s.at[pl.ds(base + full * 128, rem)])


def _deg_body(dstp_hbm, dego_hbm, idxd_v, ones_v, zb_v, deg_s):
    cid = lax.axis_index("c")
    sid = lax.axis_index("s")
    wid = sid * NC + cid
    pltpu.sync_copy(dstp_hbm.at[pl.ds(wid * RPW, RPW)], idxd_v)

    def fill(i, c):
        ones_v[i] = jnp.ones((16,), f32)
        return c

    lax.fori_loop(0, 128, fill, 0)
    _zero_rows(zb_v, 16)
    _fill_spmem(zb_v, deg_s, sid * ZPT, ZPT)
    plsc.subcore_barrier()

    def body(j, c):
        pltpu.sync_copy(ones_v, deg_s.at[idxd_v.at[j]], add=True)
        return c

    lax.fori_loop(0, RPW, body, 0)
    plsc.subcore_barrier()
    r0 = sid * OPT
    pltpu.sync_copy(deg_s.at[pl.ds(r0, OPT)], dego_hbm.at[cid, pl.ds(r0, OPT)])


_deg_call = functools.partial(
    pl.kernel,
    out_type=jax.ShapeDtypeStruct((NC, N, 16), f32),
    mesh=_mesh,
    scratch_types=[
        pltpu.VMEM((RPW, CH), i32),
        pltpu.VMEM((CH, 16), f32),
        pltpu.VMEM((CH, 16), f32),
        pltpu.VMEM_SHARED((NP, 16), f32),
    ],
)(_deg_body)


def _round_body(q_hbm, srcp_hbm, dstp_hbm, out_hbm, idxs_v, idxd_v, rows_v, part_s, sem):
    cid = lax.axis_index("c")
    sid = lax.axis_index("s")
    wid = sid * NC + cid
    pltpu.sync_copy(srcp_hbm.at[pl.ds(wid * RPW, RPW)], idxs_v)
    pltpu.sync_copy(dstp_hbm.at[pl.ds(wid * RPW, RPW)], idxd_v)
    _zero_rows(rows_v, D)
    _fill_spmem(rows_v, part_s, sid * ZPT, ZPT)
    plsc.subcore_barrier()

    def body(j, c):
        pltpu.async_copy(q_hbm.at[idxs_v.at[j]], rows_v, sem).wait()
        pltpu.sync_copy(rows_v, part_s.at[idxd_v.at[j]], add=True)
        return c

    lax.fori_loop(0, RPW, body, 0)
    plsc.subcore_barrier()
    r0 = sid * OPT
    pltpu.sync_copy(part_s.at[pl.ds(r0, OPT)], out_hbm.at[cid, pl.ds(r0, OPT)])


_round_call = functools.partial(
    pl.kernel,
    out_type=jax.ShapeDtypeStruct((NC, N, D), f32),
    mesh=_mesh,
    scratch_types=[
        pltpu.VMEM((RPW, CH), i32),
        pltpu.VMEM((RPW, CH), i32),
        pltpu.VMEM((CH, D), f32),
        pltpu.VMEM_SHARED((NP, D), f32),
        pltpu.SemaphoreType.DMA,
    ],
)(_round_body)


def _prep_body(dego_ref, x_ref, dinv_ref, q_ref):
    d = dego_ref[0, :, 0:1] + dego_ref[1, :, 0:1] + 1.0
    dv = lax.rsqrt(d)
    dinv_ref[...] = jnp.broadcast_to(dv, (BN, D))
    q_ref[...] = dv * x_ref[...]


def _prep_call(dego, x):
    return pl.pallas_call(
        _prep_body,
        grid=(NBLK,),
        in_specs=[
            pl.BlockSpec((NC, BN, 16), lambda i: (0, i, 0)),
            pl.BlockSpec((BN, D), lambda i: (i, 0)),
        ],
        out_specs=[
            pl.BlockSpec((BN, D), lambda i: (i, 0)),
            pl.BlockSpec((BN, D), lambda i: (i, 0)),
        ],
        out_shape=[
            jax.ShapeDtypeStruct((N, D), f32),
            jax.ShapeDtypeStruct((N, D), f32),
        ],
    )(dego, x)


def _mid_body(p_ref, q_ref, dinv_ref, out_ref):
    dv = dinv_ref[...]
    out_ref[...] = dv * dv * (p_ref[0] + p_ref[1] + q_ref[...])


def _mid_call(p, q, dinvb):
    return pl.pallas_call(
        _mid_body,
        grid=(NBLK,),
        in_specs=[
            pl.BlockSpec((NC, BN, D), lambda i: (0, i, 0)),
            pl.BlockSpec((BN, D), lambda i: (i, 0)),
            pl.BlockSpec((BN, D), lambda i: (i, 0)),
        ],
        out_specs=pl.BlockSpec((BN, D), lambda i: (i, 0)),
        out_shape=jax.ShapeDtypeStruct((N, D), f32),
    )(p, q, dinvb)


def _lend_body(p_ref, q_ref, dinv_ref, w_ref, b_ref, out_ref):
    dv = dinv_ref[...]
    h = dv * (p_ref[0] + p_ref[1] + q_ref[...])
    z = jnp.tanh(
        lax.dot_general(h, w_ref[...], (((1,), (1,)), ((), ())),
                        preferred_element_type=f32) + b_ref[...]
    )
    out_ref[...] = dv * z


def _lend_call(p, q, dinvb, W, b):
    return pl.pallas_call(
        _lend_body,
        grid=(NBLK,),
        in_specs=[
            pl.BlockSpec((NC, BN, D), lambda i: (0, i, 0)),
            pl.BlockSpec((BN, D), lambda i: (i, 0)),
            pl.BlockSpec((BN, D), lambda i: (i, 0)),
            pl.BlockSpec((D, D), lambda i: (0, 0)),
            pl.BlockSpec((1, D), lambda i: (0, 0)),
        ],
        out_specs=pl.BlockSpec((BN, D), lambda i: (i, 0)),
        out_shape=jax.ShapeDtypeStruct((N, D), f32),
    )(p, q, dinvb, W, b)


def _final_body(p_ref, q_ref, dinv_ref, w_ref, b_ref, ids_ref, out_ref, sums_s, cnt_s):
    i = pl.program_id(0)
    dv = dinv_ref[...]
    h = dv * (p_ref[0] + p_ref[1] + q_ref[...])
    z = jnp.tanh(
        lax.dot_general(h, w_ref[...], (((1,), (1,)), ((), ())),
                        preferred_element_type=f32) + b_ref[...]
    )
    ids = ids_ref[...]
    iota = lax.broadcasted_iota(i32, (BN, G), 1)
    m = (ids == iota).astype(f32)
    ps = lax.dot_general(m, z, (((0,), (0,)), ((), ())), preferred_element_type=f32)
    pc = lax.dot_general(m, jnp.ones((BN, D), f32), (((0,), (0,)), ((), ())),
                         preferred_element_type=f32)

    @pl.when(i == 0)
    def _():
        sums_s[...] = ps
        cnt_s[...] = pc

    @pl.when(i > 0)
    def _():
        sums_s[...] += ps
        cnt_s[...] += pc

    @pl.when(i == NBLK - 1)
    def _():
        out_ref[...] = sums_s[...] / jnp.maximum(cnt_s[...], 1.0)


def _final_call(p, q, dinvb, W, b, ids):
    return pl.pallas_call(
        _final_body,
        grid=(NBLK,),
        in_specs=[
            pl.BlockSpec((NC, BN, D), lambda i: (0, i, 0)),
            pl.BlockSpec((BN, D), lambda i: (i, 0)),
            pl.BlockSpec((BN, D), lambda i: (i, 0)),
            pl.BlockSpec((D, D), lambda i: (0, 0)),
            pl.BlockSpec((1, D), lambda i: (0, 0)),
            pl.BlockSpec((BN, 1), lambda i: (i, 0)),
        ],
        out_specs=pl.BlockSpec((G, D), lambda i: (0, 0)),
        out_shape=jax.ShapeDtypeStruct((G, D), f32),
        scratch_shapes=[
            pltpu.VMEM((G, D), f32),
            pltpu.VMEM((G, D), f32),
        ],
    )(p, q, dinvb, W, b, ids)


def kernel(x, edge_index, batch_ids, W1, b1, W2, b2):
    x = x.astype(f32)
    W1 = W1.astype(f32)
    W2 = W2.astype(f32)
    b1 = b1.astype(f32).reshape(1, D)
    b2 = b2.astype(f32).reshape(1, D)
    src = edge_index[0].astype(i32)
    dst = edge_index[1].astype(i32)
    srcp = jnp.concatenate([src, jnp.zeros((E_PAD - E,), i32)]).reshape(NROW, CH)
    dstp = jnp.concatenate([dst, jnp.full((E_PAD - E,), N, i32)]).reshape(NROW, CH)
    ids = batch_ids.astype(i32).reshape(N, 1)

    dego = _deg_call(dstp)
    dinvb, q = _prep_call(dego, x)
    p = _round_call(q, srcp, dstp)
    q = _mid_call(p, q, dinvb)
    p = _round_call(q, srcp, dstp)
    q = _lend_call(p, q, dinvb, W1, b1)
    p = _round_call(q, srcp, dstp)
    q = _mid_call(p, q, dinvb)
    p = _round_call(q, srcp, dstp)
    return _final_call(p, q, dinvb, W2, b2, ids)


# speedup vs baseline: 9.7947x; 9.7947x over previous
import functools

import jax
import jax.numpy as jnp
from jax import lax
from jax.experimental import pallas as pl
from jax.experimental.pallas import tpu as pltpu
from jax.experimental.pallas import tpu_sc as plsc

N = 10000
E = 320000
D = 128
G = 16

NC = 2
NS = 16
NW = NC * NS

CH = 128
RPW = 79
E_PAD = NW * RPW * CH
NROW = NW * RPW
NP = 10112
ZPT = NP // NS
OPT = 624

BN = 400
NBLK = N // BN

_mesh = plsc.VectorSubcoreMesh(core_axis_name="c", subcore_axis_name="s")
f32 = jnp.float32
i32 = jnp.int32


def _zero_rows(rows_v, ncols):
    nv = ncols // 16

    def outer(r, c):
        def inner(k, c2):
            rows_v[r, pl.ds(k * jnp.int32(16), 16)] = jnp.zeros((16,), f32)
            return c2

        return lax.fori_loop(jnp.int32(0), jnp.int32(nv), inner, c)

    lax.fori_loop(jnp.int32(0), jnp.int32(128), outer, 0)


def _fill_spmem(rows_v, part_s, base, nrows):
    full, rem = nrows // 128, nrows % 128
    for k in range(full):
        pltpu.sync_copy(rows_v, part_s.at[pl.ds(base + k * 128, 128)])
    if rem:
        pltpu.sync_copy(rows_v.at[pl.ds(0, rem)], part_s.at[pl.ds(base + full * 128, rem)])


def _deg_body(dstp_hbm, dego_hbm, idxd_v, ones_v, zb_v, deg_s):
    cid = lax.axis_index("c")
    sid = lax.axis_index("s")
    wid = sid * jnp.int32(NC) + cid
    pltpu.sync_copy(dstp_hbm.at[wid], idxd_v)

    def fill(i, c):
        ones_v[i] = jnp.ones((16,), f32)
        return c

    lax.fori_loop(jnp.int32(0), jnp.int32(128), fill, 0)
    _zero_rows(zb_v, 16)
    _fill_spmem(zb_v, deg_s, sid * jnp.int32(ZPT), ZPT)
    plsc.subcore_barrier()

    def body(j, c):
        pltpu.sync_copy(ones_v, deg_s.at[idxd_v.at[j]], add=True)
        return c

    lax.fori_loop(jnp.int32(0), jnp.int32(RPW), body, 0)
    plsc.subcore_barrier()
    r0 = sid * jnp.int32(OPT)

    @pl.when(sid < NS - 1)
    def _():
        pltpu.sync_copy(deg_s.at[pl.ds(r0, OPT)], dego_hbm.at[cid, pl.ds(r0, OPT)])

    @pl.when(sid == NS - 1)
    def _():
        pltpu.sync_copy(deg_s.at[pl.ds((NS - 1) * OPT, N - (NS - 1) * OPT)],
                        dego_hbm.at[cid, pl.ds((NS - 1) * OPT, N - (NS - 1) * OPT)])


_deg_call = functools.partial(
    pl.kernel,
    out_type=jax.ShapeDtypeStruct((NC, N, 16), f32),
    mesh=_mesh,
    scratch_types=[
        pltpu.VMEM((RPW, CH), i32),
        pltpu.VMEM((CH, 16), f32),
        pltpu.VMEM((CH, 16), f32),
        pltpu.VMEM_SHARED((NP, 16), f32),
    ],
)(_deg_body)


def _round_body(q_hbm, srcp_hbm, dstp_hbm, out_hbm, idxs_v, idxd_v, rows_v, part_s, sem):
    cid = lax.axis_index("c")
    sid = lax.axis_index("s")
    wid = sid * jnp.int32(NC) + cid
    pltpu.sync_copy(srcp_hbm.at[wid], idxs_v)
    pltpu.sync_copy(dstp_hbm.at[wid], idxd_v)
    _zero_rows(rows_v, D)
    _fill_spmem(rows_v, part_s, sid * jnp.int32(ZPT), ZPT)
    plsc.subcore_barrier()

    def body(j, c):
        pltpu.async_copy(q_hbm.at[idxs_v.at[j]], rows_v, sem).wait()
        pltpu.sync_copy(rows_v, part_s.at[idxd_v.at[j]], add=True)
        return c

    lax.fori_loop(jnp.int32(0), jnp.int32(RPW), body, 0)
    plsc.subcore_barrier()
    r0 = sid * jnp.int32(OPT)

    @pl.when(sid < NS - 1)
    def _():
        pltpu.sync_copy(part_s.at[pl.ds(r0, OPT)], out_hbm.at[cid, pl.ds(r0, OPT)])

    @pl.when(sid == NS - 1)
    def _():
        pltpu.sync_copy(part_s.at[pl.ds((NS - 1) * OPT, N - (NS - 1) * OPT)],
                        out_hbm.at[cid, pl.ds((NS - 1) * OPT, N - (NS - 1) * OPT)])


_round_call = functools.partial(
    pl.kernel,
    out_type=jax.ShapeDtypeStruct((NC, N, D), f32),
    mesh=_mesh,
    scratch_types=[
        pltpu.VMEM((RPW, CH), i32),
        pltpu.VMEM((RPW, CH), i32),
        pltpu.VMEM((CH, D), f32),
        pltpu.VMEM_SHARED((NP, D), f32),
        pltpu.SemaphoreType.DMA,
    ],
)(_round_body)


def _prep_body(dego_ref, x_ref, dinv_ref, q_ref):
    d = dego_ref[0, :, 0:1] + dego_ref[1, :, 0:1] + 1.0
    dv = lax.rsqrt(d)
    dinv_ref[...] = jnp.broadcast_to(dv, (BN, D))
    q_ref[...] = dv * x_ref[...]


def _prep_call(dego, x):
    return pl.pallas_call(
        _prep_body,
        grid=(NBLK,),
        in_specs=[
            pl.BlockSpec((NC, BN, 16), lambda i: (jnp.int32(0), i, jnp.int32(0))),
            pl.BlockSpec((BN, D), lambda i: (i, jnp.int32(0))),
        ],
        out_specs=[
            pl.BlockSpec((BN, D), lambda i: (i, jnp.int32(0))),
            pl.BlockSpec((BN, D), lambda i: (i, jnp.int32(0))),
        ],
        out_shape=[
            jax.ShapeDtypeStruct((N, D), f32),
            jax.ShapeDtypeStruct((N, D), f32),
        ],
    )(dego, x)


def _mid_body(p_ref, q_ref, dinv_ref, out_ref):
    dv = dinv_ref[...]
    out_ref[...] = dv * dv * (p_ref[0] + p_ref[1] + q_ref[...])


def _mid_call(p, q, dinvb):
    return pl.pallas_call(
        _mid_body,
        grid=(NBLK,),
        in_specs=[
            pl.BlockSpec((NC, BN, D), lambda i: (jnp.int32(0), i, jnp.int32(0))),
            pl.BlockSpec((BN, D), lambda i: (i, jnp.int32(0))),
            pl.BlockSpec((BN, D), lambda i: (i, jnp.int32(0))),
        ],
        out_specs=pl.BlockSpec((BN, D), lambda i: (i, jnp.int32(0))),
        out_shape=jax.ShapeDtypeStruct((N, D), f32),
    )(p, q, dinvb)


def _lend_body(p_ref, q_ref, dinv_ref, w_ref, b_ref, out_ref):
    dv = dinv_ref[...]
    h = dv * (p_ref[0] + p_ref[1] + q_ref[...])
    z = jnp.tanh(
        lax.dot_general(h, w_ref[...], (((1,), (1,)), ((), ())),
                        preferred_element_type=f32) + b_ref[...]
    )
    out_ref[...] = dv * z


def _lend_call(p, q, dinvb, W, b):
    return pl.pallas_call(
        _lend_body,
        grid=(NBLK,),
        in_specs=[
            pl.BlockSpec((NC, BN, D), lambda i: (jnp.int32(0), i, jnp.int32(0))),
            pl.BlockSpec((BN, D), lambda i: (i, jnp.int32(0))),
            pl.BlockSpec((BN, D), lambda i: (i, jnp.int32(0))),
            pl.BlockSpec((D, D), lambda i: (jnp.int32(0), jnp.int32(0))),
            pl.BlockSpec((1, D), lambda i: (jnp.int32(0), jnp.int32(0))),
        ],
        out_specs=pl.BlockSpec((BN, D), lambda i: (i, jnp.int32(0))),
        out_shape=jax.ShapeDtypeStruct((N, D), f32),
    )(p, q, dinvb, W, b)


def _final_body(p_ref, q_ref, dinv_ref, w_ref, b_ref, ids_ref, out_ref, sums_s, cnt_s):
    i = pl.program_id(0)
    dv = dinv_ref[...]
    h = dv * (p_ref[0] + p_ref[1] + q_ref[...])
    z = jnp.tanh(
        lax.dot_general(h, w_ref[...], (((1,), (1,)), ((), ())),
                        preferred_element_type=f32) + b_ref[...]
    )
    ids = ids_ref[...]
    iota = lax.broadcasted_iota(i32, (BN, G), 1)
    m = (ids == iota).astype(f32)
    ps = lax.dot_general(m, z, (((0,), (0,)), ((), ())), preferred_element_type=f32)
    pc = lax.dot_general(m, jnp.ones((BN, D), f32), (((0,), (0,)), ((), ())),
                         preferred_element_type=f32)

    @pl.when(i == 0)
    def _():
        sums_s[...] = ps
        cnt_s[...] = pc

    @pl.when(i > 0)
    def _():
        sums_s[...] += ps
        cnt_s[...] += pc

    @pl.when(i == NBLK - 1)
    def _():
        out_ref[...] = sums_s[...] / jnp.maximum(cnt_s[...], 1.0)


def _final_call(p, q, dinvb, W, b, ids):
    return pl.pallas_call(
        _final_body,
        grid=(NBLK,),
        in_specs=[
            pl.BlockSpec((NC, BN, D), lambda i: (jnp.int32(0), i, jnp.int32(0))),
            pl.BlockSpec((BN, D), lambda i: (i, jnp.int32(0))),
            pl.BlockSpec((BN, D), lambda i: (i, jnp.int32(0))),
            pl.BlockSpec((D, D), lambda i: (jnp.int32(0), jnp.int32(0))),
            pl.BlockSpec((1, D), lambda i: (jnp.int32(0), jnp.int32(0))),
            pl.BlockSpec((BN, 1), lambda i: (i, jnp.int32(0))),
        ],
        out_specs=pl.BlockSpec((G, D), lambda i: (jnp.int32(0), jnp.int32(0))),
        out_shape=jax.ShapeDtypeStruct((G, D), f32),
        scratch_shapes=[
            pltpu.VMEM((G, D), f32),
            pltpu.VMEM((G, D), f32),
        ],
    )(p, q, dinvb, W, b, ids)


def kernel(x, edge_index, batch_ids, W1, b1, W2, b2):
    x = x.astype(f32)
    W1 = W1.astype(f32)
    W2 = W2.astype(f32)
    b1 = b1.astype(f32).reshape(1, D)
    b2 = b2.astype(f32).reshape(1, D)
    src = edge_index[0].astype(i32)
    dst = edge_index[1].astype(i32)
    srcp = jnp.concatenate([src, jnp.zeros((E_PAD - E,), i32)]).reshape(NW, RPW, CH)
    dstp = jnp.concatenate([dst, jnp.full((E_PAD - E,), N, i32)]).reshape(NW, RPW, CH)
    ids = batch_ids.astype(i32).reshape(N, 1)

    dego = _deg_call(dstp)
    dinvb, q = _prep_call(dego, x)
    p = _round_call(q, srcp, dstp)
    q = _mid_call(p, q, dinvb)
    p = _round_call(q, srcp, dstp)
    q = _lend_call(p, q, dinvb, W1, b1)
    p = _round_call(q, srcp, dstp)
    q = _mid_call(p, q, dinvb)
    p = _round_call(q, srcp, dstp)
    return _final_call(p, q, dinvb, W2, b2, ids)


# speedup vs baseline: 18.7423x; 1.9135x over previous
import functools

import jax
import jax.numpy as jnp
from jax import lax
from jax.experimental import pallas as pl
from jax.experimental.pallas import tpu as pltpu
from jax.experimental.pallas import tpu_sc as plsc

N = 10000
E = 320000
D = 128
G = 16

NC = 2
NS = 16
NW = NC * NS

CH = 96
RPW = 106
E_PAD = NW * RPW * CH
NP = 10112
ZPT = NP // NS
OPT = 624

BN = 400
NBLK = N // BN

_mesh = plsc.VectorSubcoreMesh(core_axis_name="c", subcore_axis_name="s")
f32 = jnp.float32
i32 = jnp.int32


def _zero_rows(rows_v, ncols):
    nv = ncols // 16

    def outer(r, c):
        def inner(k, c2):
            rows_v[r, pl.ds(k * jnp.int32(16), 16)] = jnp.zeros((16,), f32)
            return c2

        return lax.fori_loop(jnp.int32(0), jnp.int32(nv), inner, c)

    lax.fori_loop(jnp.int32(0), jnp.int32(CH), outer, 0)


def _fill_spmem(rows_v, part_s, base, nrows):
    full, rem = nrows // CH, nrows % CH
    for k in range(full):
        pltpu.sync_copy(rows_v, part_s.at[pl.ds(base + k * CH, CH)])
    if rem:
        pltpu.sync_copy(rows_v.at[pl.ds(0, rem)], part_s.at[pl.ds(base + full * CH, rem)])


def _deg_body(dstp_hbm, dego_hbm, idxd_v, ones_v, zb_v, deg_s):
    cid = lax.axis_index("c")
    sid = lax.axis_index("s")
    wid = sid * jnp.int32(NC) + cid
    pltpu.sync_copy(dstp_hbm.at[wid], idxd_v)

    def fill(i, c):
        ones_v[i] = jnp.ones((16,), f32)
        return c

    lax.fori_loop(jnp.int32(0), jnp.int32(CH), fill, 0)
    _zero_rows(zb_v, 16)
    _fill_spmem(zb_v, deg_s, sid * jnp.int32(ZPT), ZPT)
    plsc.subcore_barrier()

    def body(j, c):
        pltpu.sync_copy(ones_v, deg_s.at[idxd_v.at[j]], add=True)
        return c

    lax.fori_loop(jnp.int32(0), jnp.int32(RPW), body, 0)
    plsc.subcore_barrier()
    r0 = sid * jnp.int32(OPT)

    @pl.when(sid < NS - 1)
    def _():
        pltpu.sync_copy(deg_s.at[pl.ds(r0, OPT)], dego_hbm.at[cid, pl.ds(r0, OPT)])

    @pl.when(sid == NS - 1)
    def _():
        pltpu.sync_copy(deg_s.at[pl.ds((NS - 1) * OPT, N - (NS - 1) * OPT)],
                        dego_hbm.at[cid, pl.ds((NS - 1) * OPT, N - (NS - 1) * OPT)])


_deg_call = functools.partial(
    pl.kernel,
    out_type=jax.ShapeDtypeStruct((NC, N, 16), f32),
    mesh=_mesh,
    scratch_types=[
        pltpu.VMEM((RPW, CH), i32),
        pltpu.VMEM((CH, 16), f32),
        pltpu.VMEM((CH, 16), f32),
        pltpu.VMEM_SHARED((NP, 16), f32),
    ],
)(_deg_body)


def _round_body(q_hbm, srcp_hbm, dstp_hbm, out_hbm, idxs_v, idxd_v, rows_a, rows_b, sem_a, sem_b, part_s):
    cid = lax.axis_index("c")
    sid = lax.axis_index("s")
    wid = sid * jnp.int32(NC) + cid
    pltpu.sync_copy(srcp_hbm.at[wid], idxs_v)
    pltpu.sync_copy(dstp_hbm.at[wid], idxd_v)

    def _src_at(j):
        return idxs_v.at[pl.ds(j * jnp.int32(CH), CH)]

    _zero_rows(rows_a, D)
    _fill_spmem(rows_a, part_s, sid * jnp.int32(ZPT), ZPT)
    plsc.subcore_barrier()

    pltpu.async_copy(q_hbm.at[_src_at(jnp.int32(0))], rows_a, sem_a)

    def body(i, c):
        j = jnp.int32(2) * i
        pltpu.make_async_copy(q_hbm.at[_src_at(j)], rows_a, sem_a).wait()
        pltpu.async_copy(q_hbm.at[_src_at(j + 1)], rows_b, sem_b)
        pltpu.sync_copy(rows_a, part_s.at[idxd_v.at[j]], add=True)
        pltpu.make_async_copy(q_hbm.at[_src_at(j + 1)], rows_b, sem_b).wait()
        pltpu.async_copy(q_hbm.at[_src_at(j + 2)], rows_a, sem_a)
        pltpu.sync_copy(rows_b, part_s.at[idxd_v.at[j + 1]], add=True)
        return c

    lax.fori_loop(jnp.int32(0), jnp.int32(RPW // 2 - 1), body, 0)
    j = jnp.int32(RPW - 2)
    pltpu.make_async_copy(q_hbm.at[_src_at(j)], rows_a, sem_a).wait()
    pltpu.async_copy(q_hbm.at[_src_at(j + 1)], rows_b, sem_b)
    pltpu.sync_copy(rows_a, part_s.at[idxd_v.at[j]], add=True)
    pltpu.make_async_copy(q_hbm.at[_src_at(j + 1)], rows_b, sem_b).wait()
    pltpu.sync_copy(rows_b, part_s.at[idxd_v.at[j + 1]], add=True)
    plsc.subcore_barrier()
    r0 = sid * jnp.int32(OPT)

    @pl.when(sid < NS - 1)
    def _():
        pltpu.sync_copy(part_s.at[pl.ds(r0, OPT)], out_hbm.at[cid, pl.ds(r0, OPT)])

    @pl.when(sid == NS - 1)
    def _():
        pltpu.sync_copy(part_s.at[pl.ds((NS - 1) * OPT, N - (NS - 1) * OPT)],
                        out_hbm.at[cid, pl.ds((NS - 1) * OPT, N - (NS - 1) * OPT)])


_round_call = functools.partial(
    pl.kernel,
    out_type=jax.ShapeDtypeStruct((NC, N, D), f32),
    mesh=_mesh,
    scratch_types=[
        pltpu.VMEM((RPW * CH,), i32),
        pltpu.VMEM((RPW, CH), i32),
        pltpu.VMEM((CH, D), f32),
        pltpu.VMEM((CH, D), f32),
        pltpu.SemaphoreType.DMA,
        pltpu.SemaphoreType.DMA,
        pltpu.VMEM_SHARED((NP, D), f32),
    ],
)(_round_body)


def _prep_body(dego_ref, x_ref, dinv_ref, q_ref):
    d = dego_ref[0, :, 0:1] + dego_ref[1, :, 0:1] + 1.0
    dv = lax.rsqrt(d)
    dinv_ref[...] = jnp.broadcast_to(dv, (BN, D))
    q_ref[...] = dv * x_ref[...]


def _prep_call(dego, x):
    return pl.pallas_call(
        _prep_body,
        grid=(NBLK,),
        in_specs=[
            pl.BlockSpec((NC, BN, 16), lambda i: (jnp.int32(0), i, jnp.int32(0))),
            pl.BlockSpec((BN, D), lambda i: (i, jnp.int32(0))),
        ],
        out_specs=[
            pl.BlockSpec((BN, D), lambda i: (i, jnp.int32(0))),
            pl.BlockSpec((BN, D), lambda i: (i, jnp.int32(0))),
        ],
        out_shape=[
            jax.ShapeDtypeStruct((N, D), f32),
            jax.ShapeDtypeStruct((N, D), f32),
        ],
    )(dego, x)


def _mid_body(p_ref, q_ref, dinv_ref, out_ref):
    dv = dinv_ref[...]
    out_ref[...] = dv * dv * (p_ref[0] + p_ref[1] + q_ref[...])


def _mid_call(p, q, dinvb):
    return pl.pallas_call(
        _mid_body,
        grid=(NBLK,),
        in_specs=[
            pl.BlockSpec((NC, BN, D), lambda i: (jnp.int32(0), i, jnp.int32(0))),
            pl.BlockSpec((BN, D), lambda i: (i, jnp.int32(0))),
            pl.BlockSpec((BN, D), lambda i: (i, jnp.int32(0))),
        ],
        out_specs=pl.BlockSpec((BN, D), lambda i: (i, jnp.int32(0))),
        out_shape=jax.ShapeDtypeStruct((N, D), f32),
    )(p, q, dinvb)


def _lend_body(p_ref, q_ref, dinv_ref, w_ref, b_ref, out_ref):
    dv = dinv_ref[...]
    h = dv * (p_ref[0] + p_ref[1] + q_ref[...])
    z = jnp.tanh(
        lax.dot_general(h, w_ref[...], (((1,), (1,)), ((), ())),
                        preferred_element_type=f32) + b_ref[...]
    )
    out_ref[...] = dv * z


def _lend_call(p, q, dinvb, W, b):
    return pl.pallas_call(
        _lend_body,
        grid=(NBLK,),
        in_specs=[
            pl.BlockSpec((NC, BN, D), lambda i: (jnp.int32(0), i, jnp.int32(0))),
            pl.BlockSpec((BN, D), lambda i: (i, jnp.int32(0))),
            pl.BlockSpec((BN, D), lambda i: (i, jnp.int32(0))),
            pl.BlockSpec((D, D), lambda i: (jnp.int32(0), jnp.int32(0))),
            pl.BlockSpec((1, D), lambda i: (jnp.int32(0), jnp.int32(0))),
        ],
        out_specs=pl.BlockSpec((BN, D), lambda i: (i, jnp.int32(0))),
        out_shape=jax.ShapeDtypeStruct((N, D), f32),
    )(p, q, dinvb, W, b)


def _final_body(p_ref, q_ref, dinv_ref, w_ref, b_ref, ids_ref, out_ref, sums_s, cnt_s):
    i = pl.program_id(0)
    dv = dinv_ref[...]
    h = dv * (p_ref[0] + p_ref[1] + q_ref[...])
    z = jnp.tanh(
        lax.dot_general(h, w_ref[...], (((1,), (1,)), ((), ())),
                        preferred_element_type=f32) + b_ref[...]
    )
    ids = ids_ref[...]
    iota = lax.broadcasted_iota(i32, (BN, G), 1)
    m = (ids == iota).astype(f32)
    ps = lax.dot_general(m, z, (((0,), (0,)), ((), ())), preferred_element_type=f32)
    pc = lax.dot_general(m, jnp.ones((BN, D), f32), (((0,), (0,)), ((), ())),
                         preferred_element_type=f32)

    @pl.when(i == 0)
    def _():
        sums_s[...] = ps
        cnt_s[...] = pc

    @pl.when(i > 0)
    def _():
        sums_s[...] += ps
        cnt_s[...] += pc

    @pl.when(i == NBLK - 1)
    def _():
        out_ref[...] = sums_s[...] / jnp.maximum(cnt_s[...], 1.0)


def _final_call(p, q, dinvb, W, b, ids):
    return pl.pallas_call(
        _final_body,
        grid=(NBLK,),
        in_specs=[
            pl.BlockSpec((NC, BN, D), lambda i: (jnp.int32(0), i, jnp.int32(0))),
            pl.BlockSpec((BN, D), lambda i: (i, jnp.int32(0))),
            pl.BlockSpec((BN, D), lambda i: (i, jnp.int32(0))),
            pl.BlockSpec((D, D), lambda i: (jnp.int32(0), jnp.int32(0))),
            pl.BlockSpec((1, D), lambda i: (jnp.int32(0), jnp.int32(0))),
            pl.BlockSpec((BN, 1), lambda i: (i, jnp.int32(0))),
        ],
        out_specs=pl.BlockSpec((G, D), lambda i: (jnp.int32(0), jnp.int32(0))),
        out_shape=jax.ShapeDtypeStruct((G, D), f32),
        scratch_shapes=[
            pltpu.VMEM((G, D), f32),
            pltpu.VMEM((G, D), f32),
        ],
    )(p, q, dinvb, W, b, ids)


def kernel(x, edge_index, batch_ids, W1, b1, W2, b2):
    x = x.astype(f32)
    W1 = W1.astype(f32)
    W2 = W2.astype(f32)
    b1 = b1.astype(f32).reshape(1, D)
    b2 = b2.astype(f32).reshape(1, D)
    src = edge_index[0].astype(i32)
    dst = edge_index[1].astype(i32)
    npad = E_PAD - E
    pad_src = (jnp.arange(npad, dtype=i32) * 131) % N
    pad_dst = N + (jnp.arange(npad, dtype=i32) % (NP - N))
    srcp = jnp.concatenate([src, pad_src]).reshape(NW, RPW * CH)
    dstp = jnp.concatenate([dst, pad_dst]).reshape(NW, RPW, CH)
    ids = batch_ids.astype(i32).reshape(N, 1)

    dego = _deg_call(dstp)
    dinvb, q = _prep_call(dego, x)
    p = _round_call(q, srcp, dstp)
    q = _mid_call(p, q, dinvb)
    p = _round_call(q, srcp, dstp)
    q = _lend_call(p, q, dinvb, W1, b1)
    p = _round_call(q, srcp, dstp)
    q = _mid_call(p, q, dinvb)
    p = _round_call(q, srcp, dstp)
    return _final_call(p, q, dinvb, W2, b2, ids)


# speedup vs baseline: 19.7894x; 1.0559x over previous
import functools

import jax
import jax.numpy as jnp
from jax import lax
from jax.experimental import pallas as pl
from jax.experimental.pallas import tpu as pltpu
from jax.experimental.pallas import tpu_sc as plsc

N = 10000
E = 320000
D = 128
G = 16

NC = 2
NS = 16
NW = NC * NS

CH = 96
RPW = 108
RPH = RPW // 2
E_PAD = NW * RPW * CH
NP = 10112
ZPT = NP // NS
OPT = 624

BN = 400
NBLK = N // BN

_mesh = plsc.VectorSubcoreMesh(core_axis_name="c", subcore_axis_name="s")
f32 = jnp.float32
i32 = jnp.int32


def _zero_rows(rows_v, ncols):
    nv = ncols // 16

    def outer(r, c):
        def inner(k, c2):
            rows_v[r, pl.ds(k * jnp.int32(16), 16)] = jnp.zeros((16,), f32)
            return c2

        return lax.fori_loop(jnp.int32(0), jnp.int32(nv), inner, c)

    lax.fori_loop(jnp.int32(0), jnp.int32(CH), outer, 0)


def _fill_spmem(rows_v, part_s, base, nrows):
    full, rem = nrows // CH, nrows % CH
    for k in range(full):
        pltpu.sync_copy(rows_v, part_s.at[pl.ds(base + k * CH, CH)])
    if rem:
        pltpu.sync_copy(rows_v.at[pl.ds(0, rem)], part_s.at[pl.ds(base + full * CH, rem)])


def _deg_body(dstp_hbm, dego_hbm, idxd_v, ones_v, zb_v, deg_s):
    cid = lax.axis_index("c")
    sid = lax.axis_index("s")
    wid = sid * jnp.int32(NC) + cid
    pltpu.sync_copy(dstp_hbm.at[wid], idxd_v)

    def fill(i, c):
        ones_v[i] = jnp.ones((16,), f32)
        return c

    lax.fori_loop(jnp.int32(0), jnp.int32(CH), fill, 0)
    _zero_rows(zb_v, 16)
    _fill_spmem(zb_v, deg_s, sid * jnp.int32(ZPT), ZPT)
    plsc.subcore_barrier()

    def body(j, c):
        pltpu.sync_copy(ones_v, deg_s.at[idxd_v.at[j]], add=True)
        return c

    lax.fori_loop(jnp.int32(0), jnp.int32(RPW), body, 0)
    plsc.subcore_barrier()
    r0 = sid * jnp.int32(OPT)

    @pl.when(sid < NS - 1)
    def _():
        pltpu.sync_copy(deg_s.at[pl.ds(r0, OPT)], dego_hbm.at[cid, pl.ds(r0, OPT)])

    @pl.when(sid == NS - 1)
    def _():
        pltpu.sync_copy(deg_s.at[pl.ds((NS - 1) * OPT, N - (NS - 1) * OPT)],
                        dego_hbm.at[cid, pl.ds((NS - 1) * OPT, N - (NS - 1) * OPT)])


_deg_call = functools.partial(
    pl.kernel,
    out_type=jax.ShapeDtypeStruct((NC, N, 16), f32),
    mesh=_mesh,
    scratch_types=[
        pltpu.VMEM((RPW, CH), i32),
        pltpu.VMEM((CH, 16), f32),
        pltpu.VMEM((CH, 16), f32),
        pltpu.VMEM_SHARED((NP, 16), f32),
    ],
)(_deg_body)


def _round_body(q_hbm, srcp_hbm, dstp_hbm, out_hbm, idxs_v, idxd_v, rows_a, rows_b, sem_a, sem_b, part_s):
    cid = lax.axis_index("c")
    sid = lax.axis_index("s")
    wid = sid * jnp.int32(NC) + cid
    pltpu.sync_copy(srcp_hbm.at[wid], idxs_v)

    def _src_at(j):
        return idxs_v.at[pl.ds(j * jnp.int32(CH), CH)]

    _zero_rows(rows_a, D)
    _fill_spmem(rows_a, part_s, sid * jnp.int32(ZPT), ZPT)
    plsc.subcore_barrier()

    pltpu.async_copy(q_hbm.at[_src_at(jnp.int32(0))], rows_a, sem_a)

    def _pair(j, jl):
        pltpu.make_async_copy(q_hbm.at[_src_at(j)], rows_a, sem_a).wait()
        pltpu.async_copy(q_hbm.at[_src_at(j + 1)], rows_b, sem_b)
        pltpu.sync_copy(rows_a, part_s.at[idxd_v.at[jl]], add=True)
        pltpu.make_async_copy(q_hbm.at[_src_at(j + 1)], rows_b, sem_b).wait()
        pltpu.async_copy(q_hbm.at[_src_at(j + 2)], rows_a, sem_a)
        pltpu.sync_copy(rows_b, part_s.at[idxd_v.at[jl + 1]], add=True)

    pltpu.sync_copy(dstp_hbm.at[wid * jnp.int32(2)], idxd_v)

    def body0(i, c):
        jl = jnp.int32(2) * i
        _pair(jl, jl)
        return c

    lax.fori_loop(jnp.int32(0), jnp.int32(RPH // 2), body0, 0)
    pltpu.sync_copy(dstp_hbm.at[wid * jnp.int32(2) + jnp.int32(1)], idxd_v)

    def body1(i, c):
        jl = jnp.int32(2) * i
        _pair(jl + jnp.int32(RPH), jl)
        return c

    lax.fori_loop(jnp.int32(0), jnp.int32(RPH // 2 - 1), body1, 0)
    j = jnp.int32(RPW - 2)
    jl = jnp.int32(RPH - 2)
    pltpu.make_async_copy(q_hbm.at[_src_at(j)], rows_a, sem_a).wait()
    pltpu.async_copy(q_hbm.at[_src_at(j + 1)], rows_b, sem_b)
    pltpu.sync_copy(rows_a, part_s.at[idxd_v.at[jl]], add=True)
    pltpu.make_async_copy(q_hbm.at[_src_at(j + 1)], rows_b, sem_b).wait()
    pltpu.sync_copy(rows_b, part_s.at[idxd_v.at[jl + 1]], add=True)
    plsc.subcore_barrier()
    r0 = sid * jnp.int32(OPT)

    @pl.when(sid < NS - 1)
    def _():
        pltpu.sync_copy(part_s.at[pl.ds(r0, OPT)], out_hbm.at[cid, pl.ds(r0, OPT)])

    @pl.when(sid == NS - 1)
    def _():
        pltpu.sync_copy(part_s.at[pl.ds((NS - 1) * OPT, N - (NS - 1) * OPT)],
                        out_hbm.at[cid, pl.ds((NS - 1) * OPT, N - (NS - 1) * OPT)])


_round_call = functools.partial(
    pl.kernel,
    out_type=jax.ShapeDtypeStruct((NC, N, D), f32),
    mesh=_mesh,
    scratch_types=[
        pltpu.VMEM((RPW * CH,), i32),
        pltpu.VMEM((RPH, CH), i32),
        pltpu.VMEM((CH, D), f32),
        pltpu.VMEM((CH, D), f32),
        pltpu.SemaphoreType.DMA,
        pltpu.SemaphoreType.DMA,
        pltpu.VMEM_SHARED((NP, D), f32),
    ],
)(_round_body)


def _prep_body(dego_ref, x_ref, dinv_ref, q_ref):
    d = dego_ref[0, :, 0:1] + dego_ref[1, :, 0:1] + 1.0
    dv = lax.rsqrt(d)
    dinv_ref[...] = jnp.broadcast_to(dv, (N, D))
    q_ref[...] = dv * x_ref[...]


def _prep_call(dego, x):
    return pl.pallas_call(
        _prep_body,
        out_shape=[
            jax.ShapeDtypeStruct((N, D), f32),
            jax.ShapeDtypeStruct((N, D), f32),
        ],
    )(dego, x)


def _mid_body(p_ref, q_ref, dinv_ref, out_ref):
    dv = dinv_ref[...]
    out_ref[...] = dv * dv * (p_ref[0] + p_ref[1] + q_ref[...])


def _mid_call(p, q, dinvb):
    return pl.pallas_call(
        _mid_body,
        out_shape=jax.ShapeDtypeStruct((N, D), f32),
    )(p, q, dinvb)


def _lend_body(p_ref, q_ref, dinv_ref, w_ref, b_ref, out_ref):
    dv = dinv_ref[...]
    h = dv * (p_ref[0] + p_ref[1] + q_ref[...])
    z = jnp.tanh(
        lax.dot_general(h, w_ref[...], (((1,), (1,)), ((), ())),
                        preferred_element_type=f32) + b_ref[...]
    )
    out_ref[...] = dv * z


def _lend_call(p, q, dinvb, W, b):
    return pl.pallas_call(
        _lend_body,
        out_shape=jax.ShapeDtypeStruct((N, D), f32),
    )(p, q, dinvb, W, b)


def _final_body(p_ref, q_ref, dinv_ref, w_ref, b_ref, ids_ref, out_ref):
    dv = dinv_ref[...]
    h = dv * (p_ref[0] + p_ref[1] + q_ref[...])
    z = jnp.tanh(
        lax.dot_general(h, w_ref[...], (((1,), (1,)), ((), ())),
                        preferred_element_type=f32) + b_ref[...]
    )
    ids = ids_ref[...]
    iota = lax.broadcasted_iota(i32, (N, G), 1)
    m = (ids == iota).astype(f32)
    ps = lax.dot_general(m, z, (((0,), (0,)), ((), ())), preferred_element_type=f32)
    pc = lax.dot_general(m, jnp.ones((N, D), f32), (((0,), (0,)), ((), ())),
                         preferred_element_type=f32)
    out_ref[...] = ps / jnp.maximum(pc, 1.0)


def _final_call(p, q, dinvb, W, b, ids):
    return pl.pallas_call(
        _final_body,
        out_shape=jax.ShapeDtypeStruct((G, D), f32),
    )(p, q, dinvb, W, b, ids)


def kernel(x, edge_index, batch_ids, W1, b1, W2, b2):
    x = x.astype(f32)
    W1 = W1.astype(f32)
    W2 = W2.astype(f32)
    b1 = b1.astype(f32).reshape(1, D)
    b2 = b2.astype(f32).reshape(1, D)
    src = edge_index[0].astype(i32)
    dst = edge_index[1].astype(i32)
    npad = E_PAD - E
    pad_src = (jnp.arange(npad, dtype=i32) * 131) % N
    pad_dst = N + (jnp.arange(npad, dtype=i32) % (NP - N))
    srcp = jnp.concatenate([src, pad_src]).reshape(NW, RPW * CH)
    dstp3 = jnp.concatenate([dst, pad_dst]).reshape(NW, RPW, CH)
    dstp = dstp3.reshape(NW * 2, RPH, CH)
    ids = batch_ids.astype(i32).reshape(N, 1)

    dego = _deg_call(dstp3)
    dinvb, q = _prep_call(dego, x)
    p = _round_call(q, srcp, dstp)
    q = _mid_call(p, q, dinvb)
    p = _round_call(q, srcp, dstp)
    q = _lend_call(p, q, dinvb, W1, b1)
    p = _round_call(q, srcp, dstp)
    q = _mid_call(p, q, dinvb)
    p = _round_call(q, srcp, dstp)
    return _final_call(p, q, dinvb, W2, b2, ids)


# speedup vs baseline: 20.1482x; 1.0181x over previous
import functools

import jax
import jax.numpy as jnp
from jax import lax
from jax.experimental import pallas as pl
from jax.experimental.pallas import tpu as pltpu
from jax.experimental.pallas import tpu_sc as plsc

N = 10000
E = 320000
D = 128
G = 16

NC = 2
NS = 16
NW = NC * NS

CH = 96
RPW = 106
E_PAD = NW * RPW * CH
NP = 10112
ZPT = NP // NS
OPT = 624

BN = 400
NBLK = N // BN

_mesh = plsc.VectorSubcoreMesh(core_axis_name="c", subcore_axis_name="s")
f32 = jnp.float32
i32 = jnp.int32


def _zero_rows(rows_v, ncols):
    nv = ncols // 16

    def outer(r, c):
        def inner(k, c2):
            rows_v[r, pl.ds(k * jnp.int32(16), 16)] = jnp.zeros((16,), f32)
            return c2

        return lax.fori_loop(jnp.int32(0), jnp.int32(nv), inner, c)

    lax.fori_loop(jnp.int32(0), jnp.int32(CH), outer, 0)


def _fill_spmem(rows_v, part_s, base, nrows):
    full, rem = nrows // CH, nrows % CH
    for k in range(full):
        pltpu.sync_copy(rows_v, part_s.at[pl.ds(base + k * CH, CH)])
    if rem:
        pltpu.sync_copy(rows_v.at[pl.ds(0, rem)], part_s.at[pl.ds(base + full * CH, rem)])


def _deg_body(dstp_hbm, dego_hbm, idxd_v, ones_v, zb_v, deg_s):
    cid = lax.axis_index("c")
    sid = lax.axis_index("s")
    wid = sid * jnp.int32(NC) + cid
    pltpu.sync_copy(dstp_hbm.at[wid], idxd_v)

    def fill(i, c):
        ones_v[i] = jnp.ones((16,), f32)
        return c

    lax.fori_loop(jnp.int32(0), jnp.int32(CH), fill, 0)
    _zero_rows(zb_v, 16)
    _fill_spmem(zb_v, deg_s, sid * jnp.int32(ZPT), ZPT)
    plsc.subcore_barrier()

    def body(j, c):
        pltpu.sync_copy(ones_v, deg_s.at[idxd_v.at[j]], add=True)
        return c

    lax.fori_loop(jnp.int32(0), jnp.int32(RPW), body, 0)
    plsc.subcore_barrier()
    r0 = sid * jnp.int32(OPT)

    @pl.when(sid < NS - 1)
    def _():
        pltpu.sync_copy(deg_s.at[pl.ds(r0, OPT)], dego_hbm.at[cid, pl.ds(r0, OPT)])

    @pl.when(sid == NS - 1)
    def _():
        pltpu.sync_copy(deg_s.at[pl.ds((NS - 1) * OPT, N - (NS - 1) * OPT)],
                        dego_hbm.at[cid, pl.ds((NS - 1) * OPT, N - (NS - 1) * OPT)])


_deg_call = functools.partial(
    pl.kernel,
    out_type=jax.ShapeDtypeStruct((NC, N, 16), f32),
    mesh=_mesh,
    scratch_types=[
        pltpu.VMEM((RPW, CH), i32),
        pltpu.VMEM((CH, 16), f32),
        pltpu.VMEM((CH, 16), f32),
        pltpu.VMEM_SHARED((NP, 16), f32),
    ],
)(_deg_body)


def _round_body(q_hbm, srcp_hbm, dstp_hbm, out_hbm, idxs_v, idxd_v, rows_a, rows_b, sem_a, sem_b, part_s):
    cid = lax.axis_index("c")
    sid = lax.axis_index("s")
    wid = sid * jnp.int32(NC) + cid
    pltpu.sync_copy(srcp_hbm.at[wid], idxs_v)

    def _src_at(j):
        return idxs_v.at[pl.ds(j * jnp.int32(CH), CH)]

    _zero_rows(rows_a, D)
    _fill_spmem(rows_a, part_s, sid * jnp.int32(ZPT), ZPT)
    plsc.subcore_barrier()

    pltpu.async_copy(q_hbm.at[_src_at(jnp.int32(0))], rows_a, sem_a)

    def _pair(j, jl):
        pltpu.make_async_copy(q_hbm.at[_src_at(j)], rows_a, sem_a).wait()
        pltpu.async_copy(q_hbm.at[_src_at(j + 1)], rows_b, sem_b)
        pltpu.sync_copy(rows_a, part_s.at[idxd_v.at[jl]], add=True)
        pltpu.make_async_copy(q_hbm.at[_src_at(j + 1)], rows_b, sem_b).wait()
        pltpu.async_copy(q_hbm.at[_src_at(j + 2)], rows_a, sem_a)
        pltpu.sync_copy(rows_b, part_s.at[idxd_v.at[jl + 1]], add=True)

    pltpu.sync_copy(dstp_hbm.at[wid], idxd_v)

    def body0(i, c):
        jl = jnp.int32(2) * i
        _pair(jl, jl)
        return c

    lax.fori_loop(jnp.int32(0), jnp.int32(RPW // 2 - 1), body0, 0)
    j = jnp.int32(RPW - 2)
    jl = j
    pltpu.make_async_copy(q_hbm.at[_src_at(j)], rows_a, sem_a).wait()
    pltpu.async_copy(q_hbm.at[_src_at(j + 1)], rows_b, sem_b)
    pltpu.sync_copy(rows_a, part_s.at[idxd_v.at[jl]], add=True)
    pltpu.make_async_copy(q_hbm.at[_src_at(j + 1)], rows_b, sem_b).wait()
    pltpu.sync_copy(rows_b, part_s.at[idxd_v.at[jl + 1]], add=True)
    plsc.subcore_barrier()
    r0 = sid * jnp.int32(OPT)

    @pl.when(sid < NS - 1)
    def _():
        pltpu.sync_copy(part_s.at[pl.ds(r0, OPT)], out_hbm.at[cid, pl.ds(r0, OPT)])

    @pl.when(sid == NS - 1)
    def _():
        pltpu.sync_copy(part_s.at[pl.ds((NS - 1) * OPT, N - (NS - 1) * OPT)],
                        out_hbm.at[cid, pl.ds((NS - 1) * OPT, N - (NS - 1) * OPT)])


_round_call = functools.partial(
    pl.kernel,
    out_type=jax.ShapeDtypeStruct((NC, N, D), f32),
    mesh=_mesh,
    scratch_types=[
        pltpu.VMEM((RPW * CH,), i32),
        pltpu.VMEM((RPW, CH), i32),
        pltpu.VMEM((CH, D), f32),
        pltpu.VMEM((CH, D), f32),
        pltpu.SemaphoreType.DMA,
        pltpu.SemaphoreType.DMA,
        pltpu.VMEM_SHARED((NP, D), f32),
    ],
)(_round_body)


def _prep_body(dego_ref, x_ref, dinv_ref, q_ref):
    d = dego_ref[0, :, 0:1] + dego_ref[1, :, 0:1] + 1.0
    dv = lax.rsqrt(d)
    dinv_ref[...] = jnp.broadcast_to(dv, (N, D))
    q_ref[...] = dv * x_ref[...]


def _prep_call(dego, x):
    return pl.pallas_call(
        _prep_body,
        out_shape=[
            jax.ShapeDtypeStruct((N, D), f32),
            jax.ShapeDtypeStruct((N, D), f32),
        ],
    )(dego, x)


def _mid_body(p_ref, q_ref, dinv_ref, out_ref):
    dv = dinv_ref[...]
    out_ref[...] = dv * dv * (p_ref[0] + p_ref[1] + q_ref[...])


def _mid_call(p, q, dinvb):
    return pl.pallas_call(
        _mid_body,
        out_shape=jax.ShapeDtypeStruct((N, D), f32),
    )(p, q, dinvb)


def _lend_body(p_ref, q_ref, dinv_ref, w_ref, b_ref, out_ref):
    dv = dinv_ref[...]
    h = dv * (p_ref[0] + p_ref[1] + q_ref[...])
    z = jnp.tanh(
        lax.dot_general(h, w_ref[...], (((1,), (1,)), ((), ())),
                        preferred_element_type=f32) + b_ref[...]
    )
    out_ref[...] = dv * z


def _lend_call(p, q, dinvb, W, b):
    return pl.pallas_call(
        _lend_body,
        out_shape=jax.ShapeDtypeStruct((N, D), f32),
    )(p, q, dinvb, W, b)


def _final_body(p_ref, q_ref, dinv_ref, w_ref, b_ref, ids_ref, out_ref):
    dv = dinv_ref[...]
    h = dv * (p_ref[0] + p_ref[1] + q_ref[...])
    z = jnp.tanh(
        lax.dot_general(h, w_ref[...], (((1,), (1,)), ((), ())),
                        preferred_element_type=f32) + b_ref[...]
    )
    ids = ids_ref[...]
    iota = lax.broadcasted_iota(i32, (N, G), 1)
    m = (ids == iota).astype(f32)
    ps = lax.dot_general(m, z, (((0,), (0,)), ((), ())), preferred_element_type=f32)
    pc = lax.dot_general(m, jnp.ones((N, D), f32), (((0,), (0,)), ((), ())),
                         preferred_element_type=f32)
    out_ref[...] = ps / jnp.maximum(pc, 1.0)


def _final_call(p, q, dinvb, W, b, ids):
    return pl.pallas_call(
        _final_body,
        out_shape=jax.ShapeDtypeStruct((G, D), f32),
    )(p, q, dinvb, W, b, ids)


def kernel(x, edge_index, batch_ids, W1, b1, W2, b2):
    x = x.astype(f32)
    W1 = W1.astype(f32)
    W2 = W2.astype(f32)
    b1 = b1.astype(f32).reshape(1, D)
    b2 = b2.astype(f32).reshape(1, D)
    src = edge_index[0].astype(i32)
    dst = edge_index[1].astype(i32)
    npad = E_PAD - E
    pad_src = (jnp.arange(npad, dtype=i32) * 131) % N
    pad_dst = N + (jnp.arange(npad, dtype=i32) % (NP - N))
    srcp = jnp.concatenate([src, pad_src]).reshape(NW, RPW * CH)
    dstp = jnp.concatenate([dst, pad_dst]).reshape(NW, RPW, CH)
    ids = batch_ids.astype(i32).reshape(N, 1)

    dego = _deg_call(dstp)
    dinvb, q = _prep_call(dego, x)
    p = _round_call(q, srcp, dstp)
    q = _mid_call(p, q, dinvb)
    p = _round_call(q, srcp, dstp)
    q = _lend_call(p, q, dinvb, W1, b1)
    p = _round_call(q, srcp, dstp)
    q = _mid_call(p, q, dinvb)
    p = _round_call(q, srcp, dstp)
    return _final_call(p, q, dinvb, W2, b2, ids)
